# baseline (speedup 1.0000x reference)
import functools

import jax
import jax.numpy as jnp
import numpy as np
from jax import lax
from jax.experimental import pallas as pl
from jax.experimental.pallas import tpu as pltpu
from jax.experimental.pallas import tpu_sc as plsc

N_NODES = 10000
D = 128
N_EDGES = 320000
N_LABEL = 320000

L = 16
NC = 2
NS = 16
NW = NC * NS

N_PAD = 10240
DEAD0 = N_NODES
N_DEAD = N_PAD - N_NODES

CHUNK = 128

DEG_CHUNKS = (2 * N_EDGES + NW * CHUNK - 1) // (NW * CHUNK)
DEG_CHUNKS = 160
E_CHUNKS = 80
EC3 = 160
HD = D // 2
ROWS_PER_SUB = N_PAD // NS
ROWS_PER_W = N_PAD // NW

@functools.cache
def _mesh():
    return plsc.VectorSubcoreMesh(
        core_axis_name="c", subcore_axis_name="s", num_cores=NC,
        num_subcores=NS)


def _wid():
    return lax.axis_index("s") * NC + lax.axis_index("c")


def _zero_vec(ref, n):
    z = jnp.zeros((L,), jnp.float32)

    def body(i, _):
        ref[pl.ds(i * L, L)] = z
        return _

    lax.fori_loop(0, n // L, body, 0)


_NBUF = 4


def _front_body(src_hbm, dst_hbm, x_hbm, y_hbm, agg_hbm, dinv_hbm,
                si, di, xv, ones_v, dbuf, dv, gsems, ssems, deg_sp, agg_sp):
    core = lax.axis_index("c")
    sub = lax.axis_index("s")
    rbase = sub * ROWS_PER_SUB
    coff = core * N_PAD
    bufs = [xv.at[pl.ds(k * CHUNK, CHUNK)] for k in range(_NBUF)]

    def zrow(r, _):
        for k in range(HD // L):
            xv[r, pl.ds(k * L, L)] = jnp.zeros((L,), jnp.float32)
        return _

    lax.fori_loop(0, ROWS_PER_SUB, zrow, 0)
    _zero_vec(dbuf, ROWS_PER_SUB)
    one = jnp.ones((L,), jnp.float32)
    for k in range(CHUNK // L):
        ones_v[pl.ds(k * L, L)] = one
    pltpu.sync_copy(dbuf, deg_sp.at[pl.ds(rbase, ROWS_PER_SUB)])
    pltpu.sync_copy(xv, agg_sp.at[pl.ds(rbase, ROWS_PER_SUB)])
    pltpu.sync_copy(src_hbm.at[sub], si)
    pltpu.sync_copy(dst_hbm.at[sub], di)
    plsc.subcore_barrier()

    def dchunk(t, _):
        j0 = 2 * t
        a0 = pltpu.async_copy(ones_v, deg_sp.at[si.at[j0]], gsems[0],
                              add=True)
        a1 = pltpu.async_copy(ones_v, deg_sp.at[di.at[j0]], gsems[1],
                              add=True)
        b0 = pltpu.async_copy(ones_v, deg_sp.at[si.at[j0 + 1]], gsems[2],
                              add=True)
        b1 = pltpu.async_copy(ones_v, deg_sp.at[di.at[j0 + 1]], gsems[3],
                              add=True)
        a0.wait()
        a1.wait()
        b0.wait()
        b1.wait()
        return _

    lax.fori_loop(0, EC3 // 2, dchunk, 0)
    plsc.subcore_barrier()

    pltpu.sync_copy(deg_sp.at[pl.ds(rbase, ROWS_PER_SUB)], dbuf)
    _xrem = N_NODES - (NS - 1) * ROWS_PER_SUB

    @pl.when(sub < NS - 1)
    def _load_x_full():
        pltpu.sync_copy(x_hbm.at[pl.ds(rbase, ROWS_PER_SUB),
                                 pl.ds(core * HD, HD)], xv)

    @pl.when(sub == NS - 1)
    def _load_x_tail():
        pltpu.sync_copy(x_hbm.at[pl.ds((NS - 1) * ROWS_PER_SUB, _xrem),
                                 pl.ds(core * HD, HD)],
                        xv.at[pl.ds(0, _xrem)])

    def newt(g0, _):
        d = dbuf[pl.ds(g0 * L, L)]
        g = jnp.full((L,), 1e-3, jnp.float32)
        for _i in range(28):
            g = g * (1.5 - 0.5 * d * g * g)
        dv[pl.ds(g0 * L, L)] = jnp.where(d > 0.5, g, 0.0)
        return _

    lax.fori_loop(0, ROWS_PER_SUB // L, newt, 0)

    def sgrp(g0, _):
        dvec = dv[pl.ds(g0 * L, L)]
        for jj in range(L):
            s = dvec[jj]
            r = g0 * L + jj
            for k in range(HD // L):
                xv[r, pl.ds(k * L, L)] = xv[r, pl.ds(k * L, L)] * s
        return _

    lax.fori_loop(0, ROWS_PER_SUB // L, sgrp, 0)
    pltpu.sync_copy(xv, y_hbm.at[pl.ds(coff + rbase, ROWS_PER_SUB)])

    @pl.when(core == 0)
    def _write_dinv():
        pltpu.sync_copy(dv, dinv_hbm.at[pl.ds(rbase, ROWS_PER_SUB)])

    def offs(j, _):
        for k in range(CHUNK // L):
            si[j, pl.ds(k * L, L)] = si[j, pl.ds(k * L, L)] + coff
        return _

    lax.fori_loop(0, EC3, offs, 0)
    plsc.subcore_barrier()

    gd = [None] * _NBUF
    sd = [None] * _NBUF
    for j in range(_NBUF):
        gd[j] = pltpu.async_copy(y_hbm.at[si.at[j]], bufs[j], gsems[j])
    for j in range(EC3):
        p = j % _NBUF
        gd[p].wait()
        sd[p] = pltpu.async_copy(bufs[p], agg_sp.at[di.at[j]], ssems[p],
                                 add=True)
        if j + _NBUF < EC3:
            sd[p].wait()
            gd[p] = pltpu.async_copy(y_hbm.at[si.at[j + _NBUF]], bufs[p],
                                     gsems[p])
    for j in range(EC3 - _NBUF, EC3):
        sd[j % _NBUF].wait()
    plsc.subcore_barrier()

    pltpu.sync_copy(agg_sp.at[pl.ds(rbase, ROWS_PER_SUB)],
                    agg_hbm.at[pl.ds(coff + rbase, ROWS_PER_SUB)])


_MM_BLK = 1024


def _mm_body(lo_ref, hi_ref, w_ref, o_ref):
    a = jnp.concatenate([lo_ref[...], hi_ref[...]], axis=1)
    o = jnp.dot(a, w_ref[...], preferred_element_type=jnp.float32)
    o_ref[...] = o.astype(jnp.bfloat16)


def _matmul(agg, W):
    nblk = N_PAD // _MM_BLK
    return pl.pallas_call(
        _mm_body,
        grid=(nblk,),
        in_specs=[
            pl.BlockSpec((_MM_BLK, HD), lambda i: (i, 0)),
            pl.BlockSpec((_MM_BLK, HD), lambda i: (i + nblk, 0)),
            pl.BlockSpec((D, D), lambda i: (0, 0)),
        ],
        out_specs=pl.BlockSpec((_MM_BLK, D), lambda i: (i, 0)),
        out_shape=jax.ShapeDtypeStruct((N_PAD, D), jnp.bfloat16),
    )(agg, agg, W)


LBL_PER_W = E_CHUNKS * CHUNK


LBL_REAL = N_LABEL // NW


def _score_body(eli_hbm, emb_hbm, dinv_hbm, out_hbm, ai, bi, dv, raa,
                rba, rab, rbb, sc_v, sa0, sa1, sb0, sb1, emb_sp):
    sub = lax.axis_index("s")
    w = _wid()
    rbase = sub * ROWS_PER_SUB
    pltpu.sync_copy(emb_hbm.at[pl.ds(rbase, ROWS_PER_SUB)],
                    emb_sp.at[pl.ds(rbase, ROWS_PER_SUB)])
    pltpu.sync_copy(eli_hbm.at[pl.ds(w * LBL_REAL, LBL_REAL)],
                    ai.at[pl.ds(0, LBL_REAL)])
    pltpu.sync_copy(eli_hbm.at[pl.ds(N_LABEL + w * LBL_REAL, LBL_REAL)],
                    bi.at[pl.ds(0, LBL_REAL)])
    iot0 = lax.iota(jnp.int32, L)
    for k in range((LBL_PER_W - LBL_REAL) // L):
        v = N_NODES + k * L + iot0
        ai[pl.ds(LBL_REAL + k * L, L)] = v
        bi[pl.ds(LBL_REAL + k * L, L)] = v
    z = jnp.zeros((L,), jnp.int32)
    for k in range(2 * CHUNK // L):
        ai[pl.ds(LBL_PER_W + k * L, L)] = z
        bi[pl.ds(LBL_PER_W + k * L, L)] = z
    pltpu.sync_copy(dinv_hbm, dv)
    plsc.subcore_barrier()

    iot = lax.iota(jnp.int32, L)

    def compute(j, ra, rb):
        def grp(g, _):
            svec = jnp.zeros((L,), jnp.float32)
            for jj in range(L):
                r = g * L + jj
                acc = jnp.zeros((L,), jnp.float32)
                for k in range(D // (2 * L)):
                    a2 = ra[r, pl.ds(k * 2 * L, 2 * L)]
                    b2 = rb[r, pl.ds(k * 2 * L, 2 * L)]
                    p2 = a2 * b2
                    plo, phi = plsc.unpack(
                        p2, format=plsc.PackFormat.INTERLEAVED)
                    acc = acc + plo
                    acc = acc + phi
                svec = jnp.where(iot == jj, jnp.sum(acc), svec)
            sc_v[pl.ds(j * CHUNK + g * L, L)] = svec
            return _

        lax.fori_loop(0, CHUNK // L, grp, 0)

        def scl(k, _):
            ga = plsc.load_gather(dv, [ai[pl.ds(j * CHUNK + k * L, L)]])
            gb = plsc.load_gather(dv, [bi[pl.ds(j * CHUNK + k * L, L)]])
            sl = pl.ds(j * CHUNK + k * L, L)
            sc_v[sl] = sc_v[sl] * ga * gb
            return _

        lax.fori_loop(0, CHUNK // L, scl, 0)

    def _wait(buf, sem):
        pltpu.make_async_copy(emb_hbm.at[pl.ds(0, CHUNK)], buf, sem).wait()

    pltpu.async_copy(emb_sp.at[ai.at[pl.ds((0) * CHUNK, CHUNK)]], raa, sa0)
    pltpu.async_copy(emb_sp.at[bi.at[pl.ds((0) * CHUNK, CHUNK)]], rba, sa1)
    pltpu.async_copy(emb_hbm.at[ai.at[pl.ds((1) * CHUNK, CHUNK)]], rab, sb0)
    pltpu.async_copy(emb_hbm.at[bi.at[pl.ds((1) * CHUNK, CHUNK)]], rbb, sb1)

    def body2(t, _):
        c0 = 2 * t
        _wait(raa, sa0)
        _wait(rba, sa1)
        compute(c0, raa, rba)
        pltpu.async_copy(emb_sp.at[ai.at[pl.ds((c0 + 2) * CHUNK, CHUNK)]], raa, sa0)
        pltpu.async_copy(emb_sp.at[bi.at[pl.ds((c0 + 2) * CHUNK, CHUNK)]], rba, sa1)
        _wait(rab, sb0)
        _wait(rbb, sb1)
        compute(c0 + 1, rab, rbb)
        pltpu.async_copy(emb_hbm.at[ai.at[pl.ds((c0 + 3) * CHUNK, CHUNK)]], rab, sb0)
        pltpu.async_copy(emb_hbm.at[bi.at[pl.ds((c0 + 3) * CHUNK, CHUNK)]], rbb, sb1)
        return _

    lax.fori_loop(0, E_CHUNKS // 2, body2, 0)
    _wait(raa, sa0)
    _wait(rba, sa1)
    _wait(rab, sb0)
    _wait(rbb, sb1)
    pltpu.sync_copy(sc_v.at[pl.ds(0, LBL_REAL)],
                    out_hbm.at[pl.ds(w * LBL_REAL, LBL_REAL)])


@functools.cache
def _build_sc_kernels():
    mesh = _mesh()
    cp = pltpu.CompilerParams(use_tc_tiling_on_sc=False,
                              needs_layout_passes=False)
    front = pl.kernel(
        _front_body,
        out_type=(jax.ShapeDtypeStruct((NC * N_PAD, HD), jnp.float32),
                  jax.ShapeDtypeStruct((NC * N_PAD, HD), jnp.float32),
                  jax.ShapeDtypeStruct((N_PAD,), jnp.float32)),
        compiler_params=cp,
        mesh=mesh,
        scratch_types=[
            pltpu.VMEM((EC3, CHUNK), jnp.int32),
            pltpu.VMEM((EC3, CHUNK), jnp.int32),
            pltpu.VMEM((ROWS_PER_SUB, HD), jnp.float32),
            pltpu.VMEM((CHUNK,), jnp.float32),
            pltpu.VMEM((ROWS_PER_SUB,), jnp.float32),
            pltpu.VMEM((ROWS_PER_SUB,), jnp.float32),
            [pltpu.SemaphoreType.DMA for _ in range(_NBUF)],
            [pltpu.SemaphoreType.DMA for _ in range(_NBUF)],
            pltpu.VMEM_SHARED((N_PAD,), jnp.float32),
            pltpu.VMEM_SHARED((N_PAD, HD), jnp.float32),
        ],
    )
    score = pl.kernel(
        _score_body,
        out_type=jax.ShapeDtypeStruct((N_LABEL,), jnp.float32),
        compiler_params=cp,
        mesh=mesh,
        scratch_types=[
            pltpu.VMEM((LBL_PER_W + 2 * CHUNK,), jnp.int32),
            pltpu.VMEM((LBL_PER_W + 2 * CHUNK,), jnp.int32),
            pltpu.VMEM((N_PAD,), jnp.float32),
            pltpu.VMEM((CHUNK, D), jnp.bfloat16),
            pltpu.VMEM((CHUNK, D), jnp.bfloat16),
            pltpu.VMEM((CHUNK, D), jnp.bfloat16),
            pltpu.VMEM((CHUNK, D), jnp.bfloat16),
            pltpu.VMEM((LBL_PER_W,), jnp.float32),
            pltpu.SemaphoreType.DMA,
            pltpu.SemaphoreType.DMA,
            pltpu.SemaphoreType.DMA,
            pltpu.SemaphoreType.DMA,
            pltpu.VMEM_SHARED((N_PAD, D), jnp.bfloat16),
        ],
    )
    return front, score


def _pad_idx(v, total):
    n = total - v.shape[0]
    pads = DEAD0 + (np.arange(n, dtype=np.int32) % N_DEAD).astype(np.int32)
    return jnp.concatenate([v, jnp.asarray(pads, dtype=jnp.int32)])


def kernel(x, edge_index, edge_label_index, W):
    x = jnp.asarray(x, jnp.float32)
    W = jnp.asarray(W, jnp.float32)
    src = edge_index[0].astype(jnp.int32)
    dst = edge_index[1].astype(jnp.int32)
    eli = edge_label_index.astype(jnp.int32).reshape(-1)

    src_s = _pad_idx(src, NS * EC3 * CHUNK).reshape(NS, EC3, CHUNK)
    dst_s = _pad_idx(dst, NS * EC3 * CHUNK).reshape(NS, EC3, CHUNK)

    k_front, k_score = _build_sc_kernels()
    _y, agg, dinv = k_front(src_s, dst_s, x)
    emb = _matmul(agg, W)
    return k_score(eli, emb, dinv)


# speedup vs baseline: 1.6113x; 1.6113x over previous
import functools

import jax
import jax.numpy as jnp
import numpy as np
from jax import lax
from jax.experimental import pallas as pl
from jax.experimental.pallas import tpu as pltpu
from jax.experimental.pallas import tpu_sc as plsc

N_NODES = 10000
D = 128
N_EDGES = 320000
N_LABEL = 320000

L = 16
NC = 2
NS = 16
NW = NC * NS

N_PAD = 10240
DEAD0 = N_NODES
N_DEAD = N_PAD - N_NODES

CHUNK = 128

DEG_CHUNKS = (2 * N_EDGES + NW * CHUNK - 1) // (NW * CHUNK)
DEG_CHUNKS = 160
E_CHUNKS = 80
EC3 = 160
HD = D // 2
ROWS_PER_SUB = N_PAD // NS
ROWS_PER_W = N_PAD // NW

@functools.cache
def _mesh():
    return plsc.VectorSubcoreMesh(
        core_axis_name="c", subcore_axis_name="s", num_cores=NC,
        num_subcores=NS)


def _wid():
    return lax.axis_index("s") * NC + lax.axis_index("c")


def _zero_vec(ref, n):
    z = jnp.zeros((L,), jnp.float32)

    def body(i, _):
        ref[pl.ds(i * L, L)] = z
        return _

    lax.fori_loop(0, n // L, body, 0)


_NBUF = 4


def _front_body(src_hbm, dst_hbm, x_hbm, y_hbm, agg_hbm, dinv_hbm,
                si, di, xv, ones_v, dbuf, dv, gsems, ssems, deg_sp, agg_sp):
    core = lax.axis_index("c")
    sub = lax.axis_index("s")
    rbase = sub * ROWS_PER_SUB
    coff = core * N_PAD
    bufs = [xv.at[pl.ds(k * CHUNK, CHUNK)] for k in range(_NBUF)]

    def zrow(r, _):
        for k in range(HD // L):
            xv[r, pl.ds(k * L, L)] = jnp.zeros((L,), jnp.float32)
        return _

    lax.fori_loop(0, ROWS_PER_SUB, zrow, 0)
    _zero_vec(dbuf, ROWS_PER_SUB)
    one = jnp.ones((L,), jnp.float32)
    for k in range(CHUNK // L):
        ones_v[pl.ds(k * L, L)] = one
    pltpu.sync_copy(dbuf, deg_sp.at[pl.ds(rbase, ROWS_PER_SUB)])
    pltpu.sync_copy(xv, agg_sp.at[pl.ds(rbase, ROWS_PER_SUB)])
    pltpu.sync_copy(src_hbm.at[sub], si)
    pltpu.sync_copy(dst_hbm.at[sub], di)
    plsc.subcore_barrier()

    def dchunk(t, _):
        j0 = 2 * t
        a0 = pltpu.async_copy(ones_v, deg_sp.at[si.at[j0]], gsems[0],
                              add=True)
        a1 = pltpu.async_copy(ones_v, deg_sp.at[di.at[j0]], gsems[1],
                              add=True)
        b0 = pltpu.async_copy(ones_v, deg_sp.at[si.at[j0 + 1]], gsems[2],
                              add=True)
        b1 = pltpu.async_copy(ones_v, deg_sp.at[di.at[j0 + 1]], gsems[3],
                              add=True)
        a0.wait()
        a1.wait()
        b0.wait()
        b1.wait()
        return _

    lax.fori_loop(0, EC3 // 2, dchunk, 0)
    plsc.subcore_barrier()

    pltpu.sync_copy(deg_sp.at[pl.ds(rbase, ROWS_PER_SUB)], dbuf)
    _xrem = N_NODES - (NS - 1) * ROWS_PER_SUB

    @pl.when(sub < NS - 1)
    def _load_x_full():
        pltpu.sync_copy(x_hbm.at[pl.ds(rbase, ROWS_PER_SUB),
                                 pl.ds(core * HD, HD)], xv)

    @pl.when(sub == NS - 1)
    def _load_x_tail():
        pltpu.sync_copy(x_hbm.at[pl.ds((NS - 1) * ROWS_PER_SUB, _xrem),
                                 pl.ds(core * HD, HD)],
                        xv.at[pl.ds(0, _xrem)])

    def newt(g0, _):
        d = dbuf[pl.ds(g0 * L, L)]
        g = jnp.full((L,), 1e-3, jnp.float32)
        for _i in range(28):
            g = g * (1.5 - 0.5 * d * g * g)
        dv[pl.ds(g0 * L, L)] = jnp.where(d > 0.5, g, 0.0)
        return _

    lax.fori_loop(0, ROWS_PER_SUB // L, newt, 0)

    def sgrp(g0, _):
        dvec = dv[pl.ds(g0 * L, L)]
        for jj in range(L):
            s = dvec[jj]
            r = g0 * L + jj
            for k in range(HD // L):
                xv[r, pl.ds(k * L, L)] = xv[r, pl.ds(k * L, L)] * s
        return _

    lax.fori_loop(0, ROWS_PER_SUB // L, sgrp, 0)
    pltpu.sync_copy(xv, y_hbm.at[pl.ds(coff + rbase, ROWS_PER_SUB)])

    @pl.when(core == 0)
    def _write_dinv():
        pltpu.sync_copy(dv, dinv_hbm.at[pl.ds(rbase, ROWS_PER_SUB)])

    def offs(j, _):
        for k in range(CHUNK // L):
            si[j, pl.ds(k * L, L)] = si[j, pl.ds(k * L, L)] + coff
        return _

    lax.fori_loop(0, EC3, offs, 0)
    plsc.subcore_barrier()

    gd = [None] * _NBUF
    sd = [None] * _NBUF
    for j in range(_NBUF):
        gd[j] = pltpu.async_copy(y_hbm.at[si.at[j]], bufs[j], gsems[j])
    for j in range(EC3):
        p = j % _NBUF
        gd[p].wait()
        sd[p] = pltpu.async_copy(bufs[p], agg_sp.at[di.at[j]], ssems[p],
                                 add=True)
        if j + _NBUF < EC3:
            sd[p].wait()
            gd[p] = pltpu.async_copy(y_hbm.at[si.at[j + _NBUF]], bufs[p],
                                     gsems[p])
    for j in range(EC3 - _NBUF, EC3):
        sd[j % _NBUF].wait()
    plsc.subcore_barrier()

    pltpu.sync_copy(agg_sp.at[pl.ds(rbase, ROWS_PER_SUB)],
                    agg_hbm.at[pl.ds(coff + rbase, ROWS_PER_SUB)])


_MM_BLK = 1024


def _mm_body(lo_ref, hi_ref, w_ref, o_ref):
    a = jnp.concatenate([lo_ref[...], hi_ref[...]], axis=1)
    o = jnp.dot(a, w_ref[...], preferred_element_type=jnp.float32)
    o_ref[...] = o.astype(jnp.bfloat16)


def _matmul(agg, W):
    nblk = N_PAD // _MM_BLK
    return pl.pallas_call(
        _mm_body,
        grid=(nblk,),
        in_specs=[
            pl.BlockSpec((_MM_BLK, HD), lambda i: (i, 0)),
            pl.BlockSpec((_MM_BLK, HD), lambda i: (i + nblk, 0)),
            pl.BlockSpec((D, D), lambda i: (0, 0)),
        ],
        out_specs=pl.BlockSpec((_MM_BLK, D), lambda i: (i, 0)),
        out_shape=jax.ShapeDtypeStruct((N_PAD, D), jnp.bfloat16),
    )(agg, agg, W)


LBL_PER_W = E_CHUNKS * CHUNK


LBL_REAL = N_LABEL // NW


def _score_body(eli_hbm, emb_hbm, dinv_hbm, out_hbm, ai, bi, dv, raa,
                rba, rab, rbb, sc_v, sa0, sa1, sb0, sb1, emb_sp):
    sub = lax.axis_index("s")
    w = _wid()
    rbase = sub * ROWS_PER_SUB
    pltpu.sync_copy(emb_hbm.at[pl.ds(rbase, ROWS_PER_SUB)],
                    emb_sp.at[pl.ds(rbase, ROWS_PER_SUB)])
    pltpu.sync_copy(eli_hbm.at[pl.ds(w * LBL_REAL, LBL_REAL)],
                    ai.at[pl.ds(0, LBL_REAL)])
    pltpu.sync_copy(eli_hbm.at[pl.ds(N_LABEL + w * LBL_REAL, LBL_REAL)],
                    bi.at[pl.ds(0, LBL_REAL)])
    iot0 = lax.iota(jnp.int32, L)
    for k in range((LBL_PER_W - LBL_REAL) // L):
        v = N_NODES + k * L + iot0
        ai[pl.ds(LBL_REAL + k * L, L)] = v
        bi[pl.ds(LBL_REAL + k * L, L)] = v
    z = jnp.zeros((L,), jnp.int32)
    for k in range(2 * CHUNK // L):
        ai[pl.ds(LBL_PER_W + k * L, L)] = z
        bi[pl.ds(LBL_PER_W + k * L, L)] = z
    pltpu.sync_copy(dinv_hbm, dv)
    plsc.subcore_barrier()

    iot = lax.iota(jnp.int32, L)

    def compute(j, ra, rb):
        def grp(g, _):
            svec = jnp.zeros((L,), jnp.float32)
            for jj in range(L):
                r = g * L + jj
                acc = jnp.zeros((L,), jnp.float32)
                for k in range(D // (2 * L)):
                    a2 = ra[r, pl.ds(k * 2 * L, 2 * L)]
                    b2 = rb[r, pl.ds(k * 2 * L, 2 * L)]
                    p2 = a2 * b2
                    plo, phi = plsc.unpack(
                        p2, format=plsc.PackFormat.INTERLEAVED)
                    acc = acc + plo
                    acc = acc + phi
                svec = jnp.where(iot == jj, jnp.sum(acc), svec)
            sc_v[pl.ds(j * CHUNK + g * L, L)] = svec
            return _

        lax.fori_loop(0, CHUNK // L, grp, 0)

        def scl(k, _):
            ga = plsc.load_gather(dv, [ai[pl.ds(j * CHUNK + k * L, L)]])
            gb = plsc.load_gather(dv, [bi[pl.ds(j * CHUNK + k * L, L)]])
            sl = pl.ds(j * CHUNK + k * L, L)
            sc_v[sl] = sc_v[sl] * ga * gb
            return _

        lax.fori_loop(0, CHUNK // L, scl, 0)

    def _wait(buf, sem):
        pltpu.make_async_copy(emb_hbm.at[pl.ds(0, CHUNK)], buf, sem).wait()

    pltpu.async_copy(emb_sp.at[ai.at[pl.ds((0) * CHUNK, CHUNK)]], raa, sa0)
    pltpu.async_copy(emb_sp.at[bi.at[pl.ds((0) * CHUNK, CHUNK)]], rba, sa1)
    pltpu.async_copy(emb_sp.at[ai.at[pl.ds((1) * CHUNK, CHUNK)]], rab, sb0)
    pltpu.async_copy(emb_sp.at[bi.at[pl.ds((1) * CHUNK, CHUNK)]], rbb, sb1)

    def body2(t, _):
        c0 = 2 * t
        _wait(raa, sa0)
        _wait(rba, sa1)
        compute(c0, raa, rba)
        pltpu.async_copy(emb_sp.at[ai.at[pl.ds((c0 + 2) * CHUNK, CHUNK)]], raa, sa0)
        pltpu.async_copy(emb_sp.at[bi.at[pl.ds((c0 + 2) * CHUNK, CHUNK)]], rba, sa1)
        _wait(rab, sb0)
        _wait(rbb, sb1)
        compute(c0 + 1, rab, rbb)
        pltpu.async_copy(emb_sp.at[ai.at[pl.ds((c0 + 3) * CHUNK, CHUNK)]], rab, sb0)
        pltpu.async_copy(emb_sp.at[bi.at[pl.ds((c0 + 3) * CHUNK, CHUNK)]], rbb, sb1)
        return _

    lax.fori_loop(0, E_CHUNKS // 2, body2, 0)
    _wait(raa, sa0)
    _wait(rba, sa1)
    _wait(rab, sb0)
    _wait(rbb, sb1)
    pltpu.sync_copy(sc_v.at[pl.ds(0, LBL_REAL)],
                    out_hbm.at[pl.ds(w * LBL_REAL, LBL_REAL)])


@functools.cache
def _build_sc_kernels():
    mesh = _mesh()
    cp = pltpu.CompilerParams(use_tc_tiling_on_sc=False,
                              needs_layout_passes=False)
    front = pl.kernel(
        _front_body,
        out_type=(jax.ShapeDtypeStruct((NC * N_PAD, HD), jnp.float32),
                  jax.ShapeDtypeStruct((NC * N_PAD, HD), jnp.float32),
                  jax.ShapeDtypeStruct((N_PAD,), jnp.float32)),
        compiler_params=cp,
        mesh=mesh,
        scratch_types=[
            pltpu.VMEM((EC3, CHUNK), jnp.int32),
            pltpu.VMEM((EC3, CHUNK), jnp.int32),
            pltpu.VMEM((ROWS_PER_SUB, HD), jnp.float32),
            pltpu.VMEM((CHUNK,), jnp.float32),
            pltpu.VMEM((ROWS_PER_SUB,), jnp.float32),
            pltpu.VMEM((ROWS_PER_SUB,), jnp.float32),
            [pltpu.SemaphoreType.DMA for _ in range(_NBUF)],
            [pltpu.SemaphoreType.DMA for _ in range(_NBUF)],
            pltpu.VMEM_SHARED((N_PAD,), jnp.float32),
            pltpu.VMEM_SHARED((N_PAD, HD), jnp.float32),
        ],
    )
    score = pl.kernel(
        _score_body,
        out_type=jax.ShapeDtypeStruct((N_LABEL,), jnp.float32),
        compiler_params=cp,
        mesh=mesh,
        scratch_types=[
            pltpu.VMEM((LBL_PER_W + 2 * CHUNK,), jnp.int32),
            pltpu.VMEM((LBL_PER_W + 2 * CHUNK,), jnp.int32),
            pltpu.VMEM((N_PAD,), jnp.float32),
            pltpu.VMEM((CHUNK, D), jnp.bfloat16),
            pltpu.VMEM((CHUNK, D), jnp.bfloat16),
            pltpu.VMEM((CHUNK, D), jnp.bfloat16),
            pltpu.VMEM((CHUNK, D), jnp.bfloat16),
            pltpu.VMEM((LBL_PER_W,), jnp.float32),
            pltpu.SemaphoreType.DMA,
            pltpu.SemaphoreType.DMA,
            pltpu.SemaphoreType.DMA,
            pltpu.SemaphoreType.DMA,
            pltpu.VMEM_SHARED((N_PAD, D), jnp.bfloat16),
        ],
    )
    return front, score


def _pad_idx(v, total):
    n = total - v.shape[0]
    pads = DEAD0 + (np.arange(n, dtype=np.int32) % N_DEAD).astype(np.int32)
    return jnp.concatenate([v, jnp.asarray(pads, dtype=jnp.int32)])


def kernel(x, edge_index, edge_label_index, W):
    x = jnp.asarray(x, jnp.float32)
    W = jnp.asarray(W, jnp.float32)
    src = edge_index[0].astype(jnp.int32)
    dst = edge_index[1].astype(jnp.int32)
    eli = edge_label_index.astype(jnp.int32).reshape(-1)

    src_s = _pad_idx(src, NS * EC3 * CHUNK).reshape(NS, EC3, CHUNK)
    dst_s = _pad_idx(dst, NS * EC3 * CHUNK).reshape(NS, EC3, CHUNK)

    k_front, k_score = _build_sc_kernels()
    _y, agg, dinv = k_front(src_s, dst_s, x)
    emb = _matmul(agg, W)
    return k_score(eli, emb, dinv)


# speedup vs baseline: 1.6396x; 1.0175x over previous
import functools

import jax
import jax.numpy as jnp
import numpy as np
from jax import lax
from jax.experimental import pallas as pl
from jax.experimental.pallas import tpu as pltpu
from jax.experimental.pallas import tpu_sc as plsc

N_NODES = 10000
D = 128
N_EDGES = 320000
N_LABEL = 320000

L = 16
NC = 2
NS = 16
NW = NC * NS

N_PAD = 10240
DEAD0 = N_NODES
N_DEAD = N_PAD - N_NODES

CHUNK = 128

DEG_CHUNKS = (2 * N_EDGES + NW * CHUNK - 1) // (NW * CHUNK)
DEG_CHUNKS = 160
E_CHUNKS = 80
EC3 = 160
HD = D // 2
ROWS_PER_SUB = N_PAD // NS
ROWS_PER_W = N_PAD // NW

@functools.cache
def _mesh():
    return plsc.VectorSubcoreMesh(
        core_axis_name="c", subcore_axis_name="s", num_cores=NC,
        num_subcores=NS)


def _wid():
    return lax.axis_index("s") * NC + lax.axis_index("c")


def _zero_vec(ref, n):
    z = jnp.zeros((L,), jnp.float32)

    def body(i, _):
        ref[pl.ds(i * L, L)] = z
        return _

    lax.fori_loop(0, n // L, body, 0)


_NBUF = 4


def _front_body(src_hbm, dst_hbm, x_hbm, y_hbm, agg_hbm, dinv_hbm,
                si, di, xv, ones_v, dbuf, dv, gsems, ssems, deg_sp, agg_sp):
    core = lax.axis_index("c")
    sub = lax.axis_index("s")
    rbase = sub * ROWS_PER_SUB
    coff = core * N_PAD
    bufs = [xv.at[pl.ds(k * CHUNK, CHUNK)] for k in range(_NBUF)]

    def zrow(r, _):
        for k in range(HD // L):
            xv[r, pl.ds(k * L, L)] = jnp.zeros((L,), jnp.float32)
        return _

    lax.fori_loop(0, ROWS_PER_SUB, zrow, 0)
    _zero_vec(dbuf, ROWS_PER_SUB)
    one = jnp.ones((L,), jnp.float32)
    for k in range(CHUNK // L):
        ones_v[pl.ds(k * L, L)] = one
    pltpu.sync_copy(dbuf, deg_sp.at[pl.ds(rbase, ROWS_PER_SUB)])
    pltpu.sync_copy(xv, agg_sp.at[pl.ds(rbase, ROWS_PER_SUB)])
    pltpu.sync_copy(src_hbm.at[sub], si)
    pltpu.sync_copy(dst_hbm.at[sub], di)
    _xrem = N_NODES - (NS - 1) * ROWS_PER_SUB

    @pl.when(sub < NS - 1)
    def _load_x_full():
        pltpu.async_copy(x_hbm.at[pl.ds(rbase, ROWS_PER_SUB),
                                  pl.ds(core * HD, HD)], xv, ssems[0])

    @pl.when(sub == NS - 1)
    def _load_x_tail():
        pltpu.async_copy(x_hbm.at[pl.ds((NS - 1) * ROWS_PER_SUB, _xrem),
                                  pl.ds(core * HD, HD)],
                         xv.at[pl.ds(0, _xrem)], ssems[0])

    plsc.subcore_barrier()

    def dchunk(t, _):
        j0 = 4 * t
        ds_ = []
        for q in range(4):
            ds_.append(pltpu.async_copy(
                ones_v, deg_sp.at[si.at[j0 + q]], gsems[q], add=True))
            ds_.append(pltpu.async_copy(
                ones_v, deg_sp.at[di.at[j0 + q]], ssems[1 + (q % 3)],
                add=True))
        for d_ in ds_:
            d_.wait()
        return _

    lax.fori_loop(0, EC3 // 4, dchunk, 0)
    plsc.subcore_barrier()

    pltpu.sync_copy(deg_sp.at[pl.ds(rbase, ROWS_PER_SUB)], dbuf)
    @pl.when(sub < NS - 1)
    def _wait_x_full():
        pltpu.make_async_copy(x_hbm.at[pl.ds(rbase, ROWS_PER_SUB),
                                       pl.ds(core * HD, HD)],
                              xv, ssems[0]).wait()

    @pl.when(sub == NS - 1)
    def _wait_x_tail():
        pltpu.make_async_copy(x_hbm.at[pl.ds((NS - 1) * ROWS_PER_SUB, _xrem),
                                       pl.ds(core * HD, HD)],
                              xv.at[pl.ds(0, _xrem)], ssems[0]).wait()

    def newt(g0, _):
        d = dbuf[pl.ds(g0 * L, L)]
        g = jnp.full((L,), 1e-3, jnp.float32)
        for _i in range(28):
            g = g * (1.5 - 0.5 * d * g * g)
        dv[pl.ds(g0 * L, L)] = jnp.where(d > 0.5, g, 0.0)
        return _

    lax.fori_loop(0, ROWS_PER_SUB // L, newt, 0)

    def sgrp(g0, _):
        dvec = dv[pl.ds(g0 * L, L)]
        for jj in range(L):
            s = dvec[jj]
            r = g0 * L + jj
            for k in range(HD // L):
                xv[r, pl.ds(k * L, L)] = xv[r, pl.ds(k * L, L)] * s
        return _

    lax.fori_loop(0, ROWS_PER_SUB // L, sgrp, 0)
    pltpu.sync_copy(xv, y_hbm.at[pl.ds(coff + rbase, ROWS_PER_SUB)])

    @pl.when(core == 0)
    def _write_dinv():
        pltpu.sync_copy(dv, dinv_hbm.at[pl.ds(rbase, ROWS_PER_SUB)])

    def offs(j, _):
        for k in range(CHUNK // L):
            si[j, pl.ds(k * L, L)] = si[j, pl.ds(k * L, L)] + coff
        return _

    lax.fori_loop(0, EC3, offs, 0)
    plsc.subcore_barrier()

    gd = [None] * _NBUF
    sd = [None] * _NBUF
    for j in range(_NBUF):
        gd[j] = pltpu.async_copy(y_hbm.at[si.at[j]], bufs[j], gsems[j])
    for j in range(EC3):
        p = j % _NBUF
        gd[p].wait()
        sd[p] = pltpu.async_copy(bufs[p], agg_sp.at[di.at[j]], ssems[p],
                                 add=True)
        if j + _NBUF < EC3:
            sd[p].wait()
            gd[p] = pltpu.async_copy(y_hbm.at[si.at[j + _NBUF]], bufs[p],
                                     gsems[p])
    for j in range(EC3 - _NBUF, EC3):
        sd[j % _NBUF].wait()
    plsc.subcore_barrier()

    pltpu.sync_copy(agg_sp.at[pl.ds(rbase, ROWS_PER_SUB)],
                    agg_hbm.at[pl.ds(coff + rbase, ROWS_PER_SUB)])


_MM_BLK = 1024


def _mm_body(lo_ref, hi_ref, w_ref, o_ref):
    a = jnp.concatenate([lo_ref[...], hi_ref[...]], axis=1)
    o = jnp.dot(a, w_ref[...], preferred_element_type=jnp.float32)
    o_ref[...] = o.astype(jnp.bfloat16)


def _matmul(agg, W):
    nblk = N_PAD // _MM_BLK
    return pl.pallas_call(
        _mm_body,
        grid=(nblk,),
        in_specs=[
            pl.BlockSpec((_MM_BLK, HD), lambda i: (i, 0)),
            pl.BlockSpec((_MM_BLK, HD), lambda i: (i + nblk, 0)),
            pl.BlockSpec((D, D), lambda i: (0, 0)),
        ],
        out_specs=pl.BlockSpec((_MM_BLK, D), lambda i: (i, 0)),
        out_shape=jax.ShapeDtypeStruct((N_PAD, D), jnp.bfloat16),
    )(agg, agg, W)


LBL_PER_W = E_CHUNKS * CHUNK


LBL_REAL = N_LABEL // NW


def _score_body(eli_hbm, emb_hbm, dinv_hbm, out_hbm, ai, bi, dv, raa,
                rba, rab, rbb, sc_v, sa0, sa1, sb0, sb1, emb_sp):
    sub = lax.axis_index("s")
    w = _wid()
    rbase = sub * ROWS_PER_SUB
    pltpu.sync_copy(emb_hbm.at[pl.ds(rbase, ROWS_PER_SUB)],
                    emb_sp.at[pl.ds(rbase, ROWS_PER_SUB)])
    pltpu.sync_copy(eli_hbm.at[pl.ds(w * LBL_REAL, LBL_REAL)],
                    ai.at[pl.ds(0, LBL_REAL)])
    pltpu.sync_copy(eli_hbm.at[pl.ds(N_LABEL + w * LBL_REAL, LBL_REAL)],
                    bi.at[pl.ds(0, LBL_REAL)])
    iot0 = lax.iota(jnp.int32, L)
    for k in range((LBL_PER_W - LBL_REAL) // L):
        v = N_NODES + k * L + iot0
        ai[pl.ds(LBL_REAL + k * L, L)] = v
        bi[pl.ds(LBL_REAL + k * L, L)] = v
    z = jnp.zeros((L,), jnp.int32)
    for k in range(2 * CHUNK // L):
        ai[pl.ds(LBL_PER_W + k * L, L)] = z
        bi[pl.ds(LBL_PER_W + k * L, L)] = z
    pltpu.sync_copy(dinv_hbm, dv)
    plsc.subcore_barrier()

    iot = lax.iota(jnp.int32, L)

    def compute(j, ra, rb):
        def grp(g, _):
            svec = jnp.zeros((L,), jnp.float32)
            for jj in range(L):
                r = g * L + jj
                acc = jnp.zeros((L,), jnp.float32)
                for k in range(D // (2 * L)):
                    a2 = ra[r, pl.ds(k * 2 * L, 2 * L)]
                    b2 = rb[r, pl.ds(k * 2 * L, 2 * L)]
                    p2 = a2 * b2
                    plo, phi = plsc.unpack(
                        p2, format=plsc.PackFormat.INTERLEAVED)
                    acc = acc + plo
                    acc = acc + phi
                svec = jnp.where(iot == jj, jnp.sum(acc), svec)
            sc_v[pl.ds(j * CHUNK + g * L, L)] = svec
            return _

        lax.fori_loop(0, CHUNK // L, grp, 0)

        def scl(k, _):
            ga = plsc.load_gather(dv, [ai[pl.ds(j * CHUNK + k * L, L)]])
            gb = plsc.load_gather(dv, [bi[pl.ds(j * CHUNK + k * L, L)]])
            sl = pl.ds(j * CHUNK + k * L, L)
            sc_v[sl] = sc_v[sl] * ga * gb
            return _

        lax.fori_loop(0, CHUNK // L, scl, 0)

    def _wait(buf, sem):
        pltpu.make_async_copy(emb_hbm.at[pl.ds(0, CHUNK)], buf, sem).wait()

    pltpu.async_copy(emb_sp.at[ai.at[pl.ds((0) * CHUNK, CHUNK)]], raa, sa0)
    pltpu.async_copy(emb_sp.at[bi.at[pl.ds((0) * CHUNK, CHUNK)]], rba, sa1)
    pltpu.async_copy(emb_sp.at[ai.at[pl.ds((1) * CHUNK, CHUNK)]], rab, sb0)
    pltpu.async_copy(emb_sp.at[bi.at[pl.ds((1) * CHUNK, CHUNK)]], rbb, sb1)

    def body2(t, _):
        c0 = 2 * t
        _wait(raa, sa0)
        _wait(rba, sa1)
        compute(c0, raa, rba)
        pltpu.async_copy(emb_sp.at[ai.at[pl.ds((c0 + 2) * CHUNK, CHUNK)]], raa, sa0)
        pltpu.async_copy(emb_sp.at[bi.at[pl.ds((c0 + 2) * CHUNK, CHUNK)]], rba, sa1)
        _wait(rab, sb0)
        _wait(rbb, sb1)
        compute(c0 + 1, rab, rbb)
        pltpu.async_copy(emb_sp.at[ai.at[pl.ds((c0 + 3) * CHUNK, CHUNK)]], rab, sb0)
        pltpu.async_copy(emb_sp.at[bi.at[pl.ds((c0 + 3) * CHUNK, CHUNK)]], rbb, sb1)
        return _

    lax.fori_loop(0, E_CHUNKS // 2, body2, 0)
    _wait(raa, sa0)
    _wait(rba, sa1)
    _wait(rab, sb0)
    _wait(rbb, sb1)
    pltpu.sync_copy(sc_v.at[pl.ds(0, LBL_REAL)],
                    out_hbm.at[pl.ds(w * LBL_REAL, LBL_REAL)])


@functools.cache
def _build_sc_kernels():
    mesh = _mesh()
    cp = pltpu.CompilerParams(use_tc_tiling_on_sc=False,
                              needs_layout_passes=False)
    front = pl.kernel(
        _front_body,
        out_type=(jax.ShapeDtypeStruct((NC * N_PAD, HD), jnp.float32),
                  jax.ShapeDtypeStruct((NC * N_PAD, HD), jnp.float32),
                  jax.ShapeDtypeStruct((N_PAD,), jnp.float32)),
        compiler_params=cp,
        mesh=mesh,
        scratch_types=[
            pltpu.VMEM((EC3, CHUNK), jnp.int32),
            pltpu.VMEM((EC3, CHUNK), jnp.int32),
            pltpu.VMEM((ROWS_PER_SUB, HD), jnp.float32),
            pltpu.VMEM((CHUNK,), jnp.float32),
            pltpu.VMEM((ROWS_PER_SUB,), jnp.float32),
            pltpu.VMEM((ROWS_PER_SUB,), jnp.float32),
            [pltpu.SemaphoreType.DMA for _ in range(_NBUF)],
            [pltpu.SemaphoreType.DMA for _ in range(_NBUF)],
            pltpu.VMEM_SHARED((N_PAD,), jnp.float32),
            pltpu.VMEM_SHARED((N_PAD, HD), jnp.float32),
        ],
    )
    score = pl.kernel(
        _score_body,
        out_type=jax.ShapeDtypeStruct((N_LABEL,), jnp.float32),
        compiler_params=cp,
        mesh=mesh,
        scratch_types=[
            pltpu.VMEM((LBL_PER_W + 2 * CHUNK,), jnp.int32),
            pltpu.VMEM((LBL_PER_W + 2 * CHUNK,), jnp.int32),
            pltpu.VMEM((N_PAD,), jnp.float32),
            pltpu.VMEM((CHUNK, D), jnp.bfloat16),
            pltpu.VMEM((CHUNK, D), jnp.bfloat16),
            pltpu.VMEM((CHUNK, D), jnp.bfloat16),
            pltpu.VMEM((CHUNK, D), jnp.bfloat16),
            pltpu.VMEM((LBL_PER_W,), jnp.float32),
            pltpu.SemaphoreType.DMA,
            pltpu.SemaphoreType.DMA,
            pltpu.SemaphoreType.DMA,
            pltpu.SemaphoreType.DMA,
            pltpu.VMEM_SHARED((N_PAD, D), jnp.bfloat16),
        ],
    )
    return front, score


def _pad_idx(v, total):
    n = total - v.shape[0]
    pads = DEAD0 + (np.arange(n, dtype=np.int32) % N_DEAD).astype(np.int32)
    return jnp.concatenate([v, jnp.asarray(pads, dtype=jnp.int32)])


def kernel(x, edge_index, edge_label_index, W):
    x = jnp.asarray(x, jnp.float32)
    W = jnp.asarray(W, jnp.float32)
    src = edge_index[0].astype(jnp.int32)
    dst = edge_index[1].astype(jnp.int32)
    eli = edge_label_index.astype(jnp.int32).reshape(-1)

    src_s = _pad_idx(src, NS * EC3 * CHUNK).reshape(NS, EC3, CHUNK)
    dst_s = _pad_idx(dst, NS * EC3 * CHUNK).reshape(NS, EC3, CHUNK)

    k_front, k_score = _build_sc_kernels()
    _y, agg, dinv = k_front(src_s, dst_s, x)
    emb = _matmul(agg, W)
    return k_score(eli, emb, dinv)


# speedup vs baseline: 1.6398x; 1.0002x over previous
import functools

import jax
import jax.numpy as jnp
import numpy as np
from jax import lax
from jax.experimental import pallas as pl
from jax.experimental.pallas import tpu as pltpu
from jax.experimental.pallas import tpu_sc as plsc

N_NODES = 10000
D = 128
N_EDGES = 320000
N_LABEL = 320000

L = 16
NC = 2
NS = 16
NW = NC * NS

N_PAD = 10240
DEAD0 = N_NODES
N_DEAD = N_PAD - N_NODES

CHUNK = 128

DEG_CHUNKS = (2 * N_EDGES + NW * CHUNK - 1) // (NW * CHUNK)
DEG_CHUNKS = 160
E_CHUNKS = 80
EC3 = 160
HD = D // 2
ROWS_PER_SUB = N_PAD // NS
ROWS_PER_W = N_PAD // NW

@functools.cache
def _mesh():
    return plsc.VectorSubcoreMesh(
        core_axis_name="c", subcore_axis_name="s", num_cores=NC,
        num_subcores=NS)


def _wid():
    return lax.axis_index("s") * NC + lax.axis_index("c")


def _zero_vec(ref, n):
    z = jnp.zeros((L,), jnp.float32)

    def body(i, _):
        ref[pl.ds(i * L, L)] = z
        return _

    lax.fori_loop(0, n // L, body, 0)


_NBUF = 4


def _front_body(src_hbm, dst_hbm, x_hbm, y_hbm, agg_hbm, dinv_hbm,
                si, di, xv, ones_v, dbuf, dv, gsems, ssems, deg_sp, agg_sp):
    core = lax.axis_index("c")
    sub = lax.axis_index("s")
    rbase = sub * ROWS_PER_SUB
    coff = core * N_PAD
    bufs = [xv.at[pl.ds(k * CHUNK, CHUNK)] for k in range(_NBUF)]

    def zrow(r, _):
        for k in range(HD // L):
            xv[r, pl.ds(k * L, L)] = jnp.zeros((L,), jnp.float32)
        return _

    lax.fori_loop(0, ROWS_PER_SUB, zrow, 0)
    _zero_vec(dbuf, ROWS_PER_SUB)
    one = jnp.ones((L,), jnp.float32)
    for k in range(CHUNK // L):
        ones_v[pl.ds(k * L, L)] = one
    pltpu.sync_copy(dbuf, deg_sp.at[pl.ds(rbase, ROWS_PER_SUB)])
    pltpu.sync_copy(xv, agg_sp.at[pl.ds(rbase, ROWS_PER_SUB)])
    pltpu.sync_copy(src_hbm.at[sub], si)
    pltpu.sync_copy(dst_hbm.at[sub], di)
    _xrem = N_NODES - (NS - 1) * ROWS_PER_SUB

    @pl.when(sub < NS - 1)
    def _load_x_full():
        pltpu.async_copy(x_hbm.at[pl.ds(rbase, ROWS_PER_SUB),
                                  pl.ds(core * HD, HD)], xv, ssems[0])

    @pl.when(sub == NS - 1)
    def _load_x_tail():
        pltpu.async_copy(x_hbm.at[pl.ds((NS - 1) * ROWS_PER_SUB, _xrem),
                                  pl.ds(core * HD, HD)],
                         xv.at[pl.ds(0, _xrem)], ssems[0])

    plsc.subcore_barrier()

    def dchunk(t, _):
        j0 = 4 * t
        ds_ = []
        for q in range(4):
            ds_.append(pltpu.async_copy(
                ones_v, deg_sp.at[si.at[j0 + q]], gsems[q], add=True))
            ds_.append(pltpu.async_copy(
                ones_v, deg_sp.at[di.at[j0 + q]], ssems[1 + (q % 3)],
                add=True))
        for d_ in ds_:
            d_.wait()
        return _

    lax.fori_loop(0, EC3 // 4, dchunk, 0)
    plsc.subcore_barrier()

    pltpu.sync_copy(deg_sp.at[pl.ds(rbase, ROWS_PER_SUB)], dbuf)
    @pl.when(sub < NS - 1)
    def _wait_x_full():
        pltpu.make_async_copy(x_hbm.at[pl.ds(rbase, ROWS_PER_SUB),
                                       pl.ds(core * HD, HD)],
                              xv, ssems[0]).wait()

    @pl.when(sub == NS - 1)
    def _wait_x_tail():
        pltpu.make_async_copy(x_hbm.at[pl.ds((NS - 1) * ROWS_PER_SUB, _xrem),
                                       pl.ds(core * HD, HD)],
                              xv.at[pl.ds(0, _xrem)], ssems[0]).wait()

    def newt(g0, _):
        d = dbuf[pl.ds(g0 * L, L)]
        g = jnp.full((L,), 1e-3, jnp.float32)
        for _i in range(28):
            g = g * (1.5 - 0.5 * d * g * g)
        dv[pl.ds(g0 * L, L)] = jnp.where(d > 0.5, g, 0.0)
        return _

    lax.fori_loop(0, ROWS_PER_SUB // L, newt, 0)

    def sgrp(g0, _):
        dvec = dv[pl.ds(g0 * L, L)]
        for jj in range(L):
            s = dvec[jj]
            r = g0 * L + jj
            for k in range(HD // L):
                xv[r, pl.ds(k * L, L)] = xv[r, pl.ds(k * L, L)] * s
        return _

    lax.fori_loop(0, ROWS_PER_SUB // L, sgrp, 0)
    pltpu.sync_copy(xv, y_hbm.at[pl.ds(coff + rbase, ROWS_PER_SUB)])

    @pl.when(core == 0)
    def _write_dinv():
        pltpu.sync_copy(dv, dinv_hbm.at[pl.ds(rbase, ROWS_PER_SUB)])

    def offs(j, _):
        for k in range(CHUNK // L):
            si[j, pl.ds(k * L, L)] = si[j, pl.ds(k * L, L)] + coff
        return _

    lax.fori_loop(0, EC3, offs, 0)
    plsc.subcore_barrier()

    gd = [None] * _NBUF
    sd = [None] * _NBUF
    for j in range(_NBUF):
        gd[j] = pltpu.async_copy(y_hbm.at[si.at[j]], bufs[j], gsems[j])
    for j in range(EC3):
        p = j % _NBUF
        gd[p].wait()
        sd[p] = pltpu.async_copy(bufs[p], agg_sp.at[di.at[j]], ssems[p],
                                 add=True)
        if j + _NBUF < EC3:
            sd[p].wait()
            gd[p] = pltpu.async_copy(y_hbm.at[si.at[j + _NBUF]], bufs[p],
                                     gsems[p])
    for j in range(EC3 - _NBUF, EC3):
        sd[j % _NBUF].wait()
    plsc.subcore_barrier()

    pltpu.sync_copy(agg_sp.at[pl.ds(rbase, ROWS_PER_SUB)],
                    agg_hbm.at[pl.ds(coff + rbase, ROWS_PER_SUB)])


_MM_BLK = 1024


def _mm_body(lo_ref, hi_ref, w_ref, o_ref):
    a = jnp.concatenate([lo_ref[...], hi_ref[...]], axis=1)
    o = jnp.dot(a, w_ref[...], preferred_element_type=jnp.float32)
    o_ref[...] = o.astype(jnp.bfloat16)


def _matmul(agg, W):
    nblk = N_PAD // _MM_BLK
    return pl.pallas_call(
        _mm_body,
        grid=(nblk,),
        in_specs=[
            pl.BlockSpec((_MM_BLK, HD), lambda i: (i, 0)),
            pl.BlockSpec((_MM_BLK, HD), lambda i: (i + nblk, 0)),
            pl.BlockSpec((D, D), lambda i: (0, 0)),
        ],
        out_specs=pl.BlockSpec((_MM_BLK, D), lambda i: (i, 0)),
        out_shape=jax.ShapeDtypeStruct((N_PAD, D), jnp.bfloat16),
    )(agg, agg, W)


LBL_PER_W = E_CHUNKS * CHUNK


LBL_REAL = N_LABEL // NW


def _score_body(eli_hbm, emb_hbm, dinv_hbm, out_hbm, ai, bi, dv, rav,
                rbv, sc_v, sav, sbv, emb_sp):
    sub = lax.axis_index("s")
    w = _wid()
    rbase = sub * ROWS_PER_SUB
    raa, rba = rav[0], rbv[0]
    pltpu.sync_copy(emb_hbm.at[pl.ds(rbase, ROWS_PER_SUB)],
                    emb_sp.at[pl.ds(rbase, ROWS_PER_SUB)])
    pltpu.sync_copy(eli_hbm.at[pl.ds(w * LBL_REAL, LBL_REAL)],
                    ai.at[pl.ds(0, LBL_REAL)])
    pltpu.sync_copy(eli_hbm.at[pl.ds(N_LABEL + w * LBL_REAL, LBL_REAL)],
                    bi.at[pl.ds(0, LBL_REAL)])
    iot0 = lax.iota(jnp.int32, L)
    for k in range((LBL_PER_W - LBL_REAL) // L):
        v = N_NODES + k * L + iot0
        ai[pl.ds(LBL_REAL + k * L, L)] = v
        bi[pl.ds(LBL_REAL + k * L, L)] = v
    z = jnp.zeros((L,), jnp.int32)
    for k in range(4 * CHUNK // L):
        ai[pl.ds(LBL_PER_W + k * L, L)] = z
        bi[pl.ds(LBL_PER_W + k * L, L)] = z
    pltpu.sync_copy(dinv_hbm, dv)
    plsc.subcore_barrier()

    iot = lax.iota(jnp.int32, L)

    def compute(j, ra, rb):
        def grp(g, _):
            svec = jnp.zeros((L,), jnp.float32)
            for jj in range(L):
                r = g * L + jj
                acc = jnp.zeros((L,), jnp.float32)
                for k in range(D // (2 * L)):
                    a2 = ra[r, pl.ds(k * 2 * L, 2 * L)]
                    b2 = rb[r, pl.ds(k * 2 * L, 2 * L)]
                    p2 = a2 * b2
                    plo, phi = plsc.unpack(
                        p2, format=plsc.PackFormat.INTERLEAVED)
                    acc = acc + plo
                    acc = acc + phi
                svec = jnp.where(iot == jj, jnp.sum(acc), svec)
            sc_v[pl.ds(j * CHUNK + g * L, L)] = svec
            return _

        lax.fori_loop(0, CHUNK // L, grp, 0)

        def scl(k, _):
            ga = plsc.load_gather(dv, [ai[pl.ds(j * CHUNK + k * L, L)]])
            gb = plsc.load_gather(dv, [bi[pl.ds(j * CHUNK + k * L, L)]])
            sl = pl.ds(j * CHUNK + k * L, L)
            sc_v[sl] = sc_v[sl] * ga * gb
            return _

        lax.fori_loop(0, CHUNK // L, scl, 0)

    def _wait(buf, sem):
        pltpu.make_async_copy(emb_hbm.at[pl.ds(0, CHUNK)], buf, sem).wait()

    KB = 2

    def _issue(c, p):
        pltpu.async_copy(emb_sp.at[ai.at[pl.ds(c * CHUNK, CHUNK)]], rav[p],
                         sav[p])
        pltpu.async_copy(emb_sp.at[bi.at[pl.ds(c * CHUNK, CHUNK)]], rbv[p],
                         sbv[p])

    for p in range(KB):
        _issue(p, p)

    def bodyk(t, _):
        c0 = KB * t
        for p in range(KB):
            _wait(rav[p], sav[p])
            _wait(rbv[p], sbv[p])
            compute(c0 + p, rav[p], rbv[p])
            _issue(c0 + p + KB, p)
        return _

    lax.fori_loop(0, E_CHUNKS // KB, bodyk, 0)
    for p in range(KB):
        _wait(rav[p], sav[p])
        _wait(rbv[p], sbv[p])
    pltpu.sync_copy(sc_v.at[pl.ds(0, LBL_REAL)],
                    out_hbm.at[pl.ds(w * LBL_REAL, LBL_REAL)])


@functools.cache
def _build_sc_kernels():
    mesh = _mesh()
    cp = pltpu.CompilerParams(use_tc_tiling_on_sc=False,
                              needs_layout_passes=False)
    front = pl.kernel(
        _front_body,
        out_type=(jax.ShapeDtypeStruct((NC * N_PAD, HD), jnp.float32),
                  jax.ShapeDtypeStruct((NC * N_PAD, HD), jnp.float32),
                  jax.ShapeDtypeStruct((N_PAD,), jnp.float32)),
        compiler_params=cp,
        mesh=mesh,
        scratch_types=[
            pltpu.VMEM((EC3, CHUNK), jnp.int32),
            pltpu.VMEM((EC3, CHUNK), jnp.int32),
            pltpu.VMEM((ROWS_PER_SUB, HD), jnp.float32),
            pltpu.VMEM((CHUNK,), jnp.float32),
            pltpu.VMEM((ROWS_PER_SUB,), jnp.float32),
            pltpu.VMEM((ROWS_PER_SUB,), jnp.float32),
            [pltpu.SemaphoreType.DMA for _ in range(_NBUF)],
            [pltpu.SemaphoreType.DMA for _ in range(_NBUF)],
            pltpu.VMEM_SHARED((N_PAD,), jnp.float32),
            pltpu.VMEM_SHARED((N_PAD, HD), jnp.float32),
        ],
    )
    score = pl.kernel(
        _score_body,
        out_type=jax.ShapeDtypeStruct((N_LABEL,), jnp.float32),
        compiler_params=cp,
        mesh=mesh,
        scratch_types=[
            pltpu.VMEM((LBL_PER_W + 4 * CHUNK,), jnp.int32),
            pltpu.VMEM((LBL_PER_W + 4 * CHUNK,), jnp.int32),
            pltpu.VMEM((N_PAD,), jnp.float32),
            [pltpu.VMEM((CHUNK, D), jnp.bfloat16) for _ in range(2)],
            [pltpu.VMEM((CHUNK, D), jnp.bfloat16) for _ in range(2)],
            pltpu.VMEM((LBL_PER_W,), jnp.float32),
            [pltpu.SemaphoreType.DMA for _ in range(2)],
            [pltpu.SemaphoreType.DMA for _ in range(2)],
            pltpu.VMEM_SHARED((N_PAD, D), jnp.bfloat16),
        ],
    )
    return front, score


def _pad_idx(v, total):
    n = total - v.shape[0]
    pads = DEAD0 + (np.arange(n, dtype=np.int32) % N_DEAD).astype(np.int32)
    return jnp.concatenate([v, jnp.asarray(pads, dtype=jnp.int32)])


def kernel(x, edge_index, edge_label_index, W):
    x = jnp.asarray(x, jnp.float32)
    W = jnp.asarray(W, jnp.float32)
    src = edge_index[0].astype(jnp.int32)
    dst = edge_index[1].astype(jnp.int32)
    eli = edge_label_index.astype(jnp.int32).reshape(-1)

    src_s = _pad_idx(src, NS * EC3 * CHUNK).reshape(NS, EC3, CHUNK)
    dst_s = _pad_idx(dst, NS * EC3 * CHUNK).reshape(NS, EC3, CHUNK)

    k_front, k_score = _build_sc_kernels()
    _y, agg, dinv = k_front(src_s, dst_s, x)
    emb = _matmul(agg, W)
    return k_score(eli, emb, dinv)


# speedup vs baseline: 1.6400x; 1.0001x over previous
import functools

import jax
import jax.numpy as jnp
import numpy as np
from jax import lax
from jax.experimental import pallas as pl
from jax.experimental.pallas import tpu as pltpu
from jax.experimental.pallas import tpu_sc as plsc

N_NODES = 10000
D = 128
N_EDGES = 320000
N_LABEL = 320000

L = 16
NC = 2
NS = 16
NW = NC * NS

N_PAD = 10240
DEAD0 = N_NODES
N_DEAD = N_PAD - N_NODES

CHUNK = 128

DEG_CHUNKS = (2 * N_EDGES + NW * CHUNK - 1) // (NW * CHUNK)
DEG_CHUNKS = 160
E_CHUNKS = 80
EC3 = 160
HD = D // 2
ROWS_PER_SUB = N_PAD // NS
ROWS_PER_W = N_PAD // NW

@functools.cache
def _mesh():
    return plsc.VectorSubcoreMesh(
        core_axis_name="c", subcore_axis_name="s", num_cores=NC,
        num_subcores=NS)


def _wid():
    return lax.axis_index("s") * NC + lax.axis_index("c")


def _zero_vec(ref, n):
    z = jnp.zeros((L,), jnp.float32)

    def body(i, _):
        ref[pl.ds(i * L, L)] = z
        return _

    lax.fori_loop(0, n // L, body, 0)


_NBUF = 4


def _front_body(src_hbm, dst_hbm, x_hbm, y_hbm, agg_hbm, dinv_hbm,
                si, di, xv, ones_v, dbuf, dv, gsems, ssems, deg_sp, agg_sp):
    core = lax.axis_index("c")
    sub = lax.axis_index("s")
    rbase = sub * ROWS_PER_SUB
    coff = core * N_PAD
    bufs = [xv.at[pl.ds(k * CHUNK, CHUNK)] for k in range(_NBUF)]

    def zrow(r, _):
        for k in range(HD // L):
            xv[r, pl.ds(k * L, L)] = jnp.zeros((L,), jnp.float32)
        return _

    lax.fori_loop(0, ROWS_PER_SUB, zrow, 0)
    _zero_vec(dbuf, ROWS_PER_SUB)
    one = jnp.ones((L,), jnp.float32)
    for k in range(CHUNK // L):
        ones_v[pl.ds(k * L, L)] = one
    plsc.subcore_barrier()
    pltpu.sync_copy(dbuf, deg_sp.at[pl.ds(rbase, ROWS_PER_SUB)])
    pltpu.sync_copy(xv, agg_sp.at[pl.ds(rbase, ROWS_PER_SUB)])
    pltpu.sync_copy(src_hbm.at[sub], si)
    pltpu.sync_copy(dst_hbm.at[sub], di)
    _xrem = N_NODES - (NS - 1) * ROWS_PER_SUB

    @pl.when(sub < NS - 1)
    def _load_x_full():
        pltpu.async_copy(x_hbm.at[pl.ds(rbase, ROWS_PER_SUB),
                                  pl.ds(core * HD, HD)], xv, ssems[0])

    @pl.when(sub == NS - 1)
    def _load_x_tail():
        pltpu.async_copy(x_hbm.at[pl.ds((NS - 1) * ROWS_PER_SUB, _xrem),
                                  pl.ds(core * HD, HD)],
                         xv.at[pl.ds(0, _xrem)], ssems[0])

    plsc.subcore_barrier()

    def dchunk(t, _):
        j0 = 4 * t
        ds_ = []
        for q in range(4):
            ds_.append(pltpu.async_copy(
                ones_v, deg_sp.at[si.at[j0 + q]], gsems[q], add=True))
            ds_.append(pltpu.async_copy(
                ones_v, deg_sp.at[di.at[j0 + q]], ssems[1 + (q % 3)],
                add=True))
        for d_ in ds_:
            d_.wait()
        return _

    lax.fori_loop(0, EC3 // 4, dchunk, 0)
    plsc.subcore_barrier()

    pltpu.sync_copy(deg_sp.at[pl.ds(rbase, ROWS_PER_SUB)], dbuf)
    @pl.when(sub < NS - 1)
    def _wait_x_full():
        pltpu.make_async_copy(x_hbm.at[pl.ds(rbase, ROWS_PER_SUB),
                                       pl.ds(core * HD, HD)],
                              xv, ssems[0]).wait()

    @pl.when(sub == NS - 1)
    def _wait_x_tail():
        pltpu.make_async_copy(x_hbm.at[pl.ds((NS - 1) * ROWS_PER_SUB, _xrem),
                                       pl.ds(core * HD, HD)],
                              xv.at[pl.ds(0, _xrem)], ssems[0]).wait()

    def newt(g0, _):
        d = dbuf[pl.ds(g0 * L, L)]
        g = jnp.full((L,), 1e-3, jnp.float32)
        for _i in range(28):
            g = g * (1.5 - 0.5 * d * g * g)
        dv[pl.ds(g0 * L, L)] = jnp.where(d > 0.5, g, 0.0)
        return _

    lax.fori_loop(0, ROWS_PER_SUB // L, newt, 0)

    def sgrp(g0, _):
        dvec = dv[pl.ds(g0 * L, L)]
        for jj in range(L):
            s = dvec[jj]
            r = g0 * L + jj
            for k in range(HD // L):
                xv[r, pl.ds(k * L, L)] = xv[r, pl.ds(k * L, L)] * s
        return _

    lax.fori_loop(0, ROWS_PER_SUB // L, sgrp, 0)
    pltpu.sync_copy(xv, y_hbm.at[pl.ds(coff + rbase, ROWS_PER_SUB)])

    @pl.when(core == 0)
    def _write_dinv():
        pltpu.sync_copy(dv, dinv_hbm.at[pl.ds(rbase, ROWS_PER_SUB)])

    def offs(j, _):
        for k in range(CHUNK // L):
            si[j, pl.ds(k * L, L)] = si[j, pl.ds(k * L, L)] + coff
        return _

    lax.fori_loop(0, EC3, offs, 0)
    plsc.subcore_barrier()

    gd = [None] * _NBUF
    sd = [None] * _NBUF
    for j in range(_NBUF):
        gd[j] = pltpu.async_copy(y_hbm.at[si.at[j]], bufs[j], gsems[j])
    for j in range(EC3):
        p = j % _NBUF
        gd[p].wait()
        sd[p] = pltpu.async_copy(bufs[p], agg_sp.at[di.at[j]], ssems[p],
                                 add=True)
        if j + _NBUF < EC3:
            sd[p].wait()
            gd[p] = pltpu.async_copy(y_hbm.at[si.at[j + _NBUF]], bufs[p],
                                     gsems[p])
    for j in range(EC3 - _NBUF, EC3):
        sd[j % _NBUF].wait()
    plsc.subcore_barrier()

    pltpu.sync_copy(agg_sp.at[pl.ds(rbase, ROWS_PER_SUB)],
                    agg_hbm.at[pl.ds(coff + rbase, ROWS_PER_SUB)])


_MM_BLK = 1024


def _mm_body(lo_ref, hi_ref, w_ref, o_ref):
    a = jnp.concatenate([lo_ref[...], hi_ref[...]], axis=1)
    o = jnp.dot(a, w_ref[...], preferred_element_type=jnp.float32)
    o_ref[...] = o.astype(jnp.bfloat16)


def _matmul(agg, W):
    nblk = N_PAD // _MM_BLK
    return pl.pallas_call(
        _mm_body,
        grid=(nblk,),
        in_specs=[
            pl.BlockSpec((_MM_BLK, HD), lambda i: (i, 0)),
            pl.BlockSpec((_MM_BLK, HD), lambda i: (i + nblk, 0)),
            pl.BlockSpec((D, D), lambda i: (0, 0)),
        ],
        out_specs=pl.BlockSpec((_MM_BLK, D), lambda i: (i, 0)),
        out_shape=jax.ShapeDtypeStruct((N_PAD, D), jnp.bfloat16),
    )(agg, agg, W)


LBL_PER_W = E_CHUNKS * CHUNK


LBL_REAL = N_LABEL // NW


def _score_body(eli_hbm, emb_hbm, dinv_hbm, out_hbm, ai, bi, dv, rav,
                rbv, sc_v, sav, sbv, emb_sp):
    sub = lax.axis_index("s")
    w = _wid()
    rbase = sub * ROWS_PER_SUB
    raa, rba = rav[0], rbv[0]
    pltpu.sync_copy(emb_hbm.at[pl.ds(rbase, ROWS_PER_SUB)],
                    emb_sp.at[pl.ds(rbase, ROWS_PER_SUB)])
    pltpu.sync_copy(eli_hbm.at[pl.ds(w * LBL_REAL, LBL_REAL)],
                    ai.at[pl.ds(0, LBL_REAL)])
    pltpu.sync_copy(eli_hbm.at[pl.ds(N_LABEL + w * LBL_REAL, LBL_REAL)],
                    bi.at[pl.ds(0, LBL_REAL)])
    iot0 = lax.iota(jnp.int32, L)
    for k in range((LBL_PER_W - LBL_REAL) // L):
        v = N_NODES + k * L + iot0
        ai[pl.ds(LBL_REAL + k * L, L)] = v
        bi[pl.ds(LBL_REAL + k * L, L)] = v
    z = jnp.zeros((L,), jnp.int32)
    for k in range(4 * CHUNK // L):
        ai[pl.ds(LBL_PER_W + k * L, L)] = z
        bi[pl.ds(LBL_PER_W + k * L, L)] = z
    pltpu.sync_copy(dinv_hbm, dv)
    plsc.subcore_barrier()

    iot = lax.iota(jnp.int32, L)

    def compute(j, ra, rb):
        def grp(g, _):
            svec = jnp.zeros((L,), jnp.float32)
            for jj in range(L):
                r = g * L + jj
                acc = jnp.zeros((L,), jnp.float32)
                for k in range(D // (2 * L)):
                    a2 = ra[r, pl.ds(k * 2 * L, 2 * L)]
                    b2 = rb[r, pl.ds(k * 2 * L, 2 * L)]
                    p2 = a2 * b2
                    plo, phi = plsc.unpack(
                        p2, format=plsc.PackFormat.INTERLEAVED)
                    acc = acc + plo
                    acc = acc + phi
                svec = jnp.where(iot == jj, jnp.sum(acc), svec)
            sc_v[pl.ds(j * CHUNK + g * L, L)] = svec
            return _

        lax.fori_loop(0, CHUNK // L, grp, 0)

        def scl(k, _):
            ga = plsc.load_gather(dv, [ai[pl.ds(j * CHUNK + k * L, L)]])
            gb = plsc.load_gather(dv, [bi[pl.ds(j * CHUNK + k * L, L)]])
            sl = pl.ds(j * CHUNK + k * L, L)
            sc_v[sl] = sc_v[sl] * ga * gb
            return _

        lax.fori_loop(0, CHUNK // L, scl, 0)

    def _wait(buf, sem):
        pltpu.make_async_copy(emb_hbm.at[pl.ds(0, CHUNK)], buf, sem).wait()

    KB = 2

    def _issue(c, p):
        pltpu.async_copy(emb_sp.at[ai.at[pl.ds(c * CHUNK, CHUNK)]], rav[p],
                         sav[p])
        pltpu.async_copy(emb_sp.at[bi.at[pl.ds(c * CHUNK, CHUNK)]], rbv[p],
                         sbv[p])

    for p in range(KB):
        _issue(p, p)

    def bodyk(t, _):
        c0 = KB * t
        for p in range(KB):
            _wait(rav[p], sav[p])
            _wait(rbv[p], sbv[p])
            compute(c0 + p, rav[p], rbv[p])
            _issue(c0 + p + KB, p)
        return _

    lax.fori_loop(0, E_CHUNKS // KB, bodyk, 0)
    for p in range(KB):
        _wait(rav[p], sav[p])
        _wait(rbv[p], sbv[p])
    pltpu.sync_copy(sc_v.at[pl.ds(0, LBL_REAL)],
                    out_hbm.at[pl.ds(w * LBL_REAL, LBL_REAL)])


@functools.cache
def _build_sc_kernels():
    mesh = _mesh()
    cp = pltpu.CompilerParams(use_tc_tiling_on_sc=False,
                              needs_layout_passes=False)
    front = pl.kernel(
        _front_body,
        out_type=(jax.ShapeDtypeStruct((NC * N_PAD, HD), jnp.float32),
                  jax.ShapeDtypeStruct((NC * N_PAD, HD), jnp.float32),
                  jax.ShapeDtypeStruct((N_PAD,), jnp.float32)),
        compiler_params=cp,
        mesh=mesh,
        scratch_types=[
            pltpu.VMEM((EC3, CHUNK), jnp.int32),
            pltpu.VMEM((EC3, CHUNK), jnp.int32),
            pltpu.VMEM((ROWS_PER_SUB, HD), jnp.float32),
            pltpu.VMEM((CHUNK,), jnp.float32),
            pltpu.VMEM((ROWS_PER_SUB,), jnp.float32),
            pltpu.VMEM((ROWS_PER_SUB,), jnp.float32),
            [pltpu.SemaphoreType.DMA for _ in range(_NBUF)],
            [pltpu.SemaphoreType.DMA for _ in range(_NBUF)],
            pltpu.VMEM_SHARED((N_PAD,), jnp.float32),
            pltpu.VMEM_SHARED((N_PAD, HD), jnp.float32),
        ],
    )
    score = pl.kernel(
        _score_body,
        out_type=jax.ShapeDtypeStruct((N_LABEL,), jnp.float32),
        compiler_params=cp,
        mesh=mesh,
        scratch_types=[
            pltpu.VMEM((LBL_PER_W + 4 * CHUNK,), jnp.int32),
            pltpu.VMEM((LBL_PER_W + 4 * CHUNK,), jnp.int32),
            pltpu.VMEM((N_PAD,), jnp.float32),
            [pltpu.VMEM((CHUNK, D), jnp.bfloat16) for _ in range(2)],
            [pltpu.VMEM((CHUNK, D), jnp.bfloat16) for _ in range(2)],
            pltpu.VMEM((LBL_PER_W,), jnp.float32),
            [pltpu.SemaphoreType.DMA for _ in range(2)],
            [pltpu.SemaphoreType.DMA for _ in range(2)],
            pltpu.VMEM_SHARED((N_PAD, D), jnp.bfloat16),
        ],
    )
    return front, score


def _pad_idx(v, total):
    n = total - v.shape[0]
    pads = DEAD0 + (np.arange(n, dtype=np.int32) % N_DEAD).astype(np.int32)
    return jnp.concatenate([v, jnp.asarray(pads, dtype=jnp.int32)])


def kernel(x, edge_index, edge_label_index, W):
    x = jnp.asarray(x, jnp.float32)
    W = jnp.asarray(W, jnp.float32)
    src = edge_index[0].astype(jnp.int32)
    dst = edge_index[1].astype(jnp.int32)
    eli = edge_label_index.astype(jnp.int32).reshape(-1)

    src_s = _pad_idx(src, NS * EC3 * CHUNK).reshape(NS, EC3, CHUNK)
    dst_s = _pad_idx(dst, NS * EC3 * CHUNK).reshape(NS, EC3, CHUNK)

    k_front, k_score = _build_sc_kernels()
    _y, agg, dinv = k_front(src_s, dst_s, x)
    emb = _matmul(agg, W)
    return k_score(eli, emb, dinv)
